# baseline (speedup 1.0000x reference)
import functools

import jax
import jax.numpy as jnp
from jax import lax
from jax.experimental import pallas as pl
from jax.experimental.pallas import tpu as pltpu
from jax.experimental.pallas import tpu_sc as plsc

NC = 2
NS = 16
NW = NC * NS

CH = 512
NB = 2


def _gather_kernel(b_per_w, n_chunks, D, idx_hbm, table_hbm, out_hbm,
                   idx_v, rows_v, gsem, ssem):
    wid = lax.axis_index("s") * NC + lax.axis_index("c")
    base = wid * b_per_w
    pltpu.sync_copy(idx_hbm.at[pl.ds(base, b_per_w)], idx_v)

    def outer(o, carry):
        g0 = o * NB
        for b in range(NB):
            off = (g0 + b) * CH
            pltpu.async_copy(table_hbm.at[idx_v.at[pl.ds(off, CH)]],
                             rows_v.at[b], gsem)
        for b in range(NB):
            off = (g0 + b) * CH
            pltpu.make_async_copy(table_hbm.at[idx_v.at[pl.ds(off, CH)]],
                                  rows_v.at[b], gsem).wait()
            pltpu.async_copy(rows_v.at[b],
                             out_hbm.at[pl.ds(base + off, CH)], ssem)
        for b in range(NB):
            off = (g0 + b) * CH
            pltpu.make_async_copy(rows_v.at[b],
                                  out_hbm.at[pl.ds(base + off, CH)],
                                  ssem).wait()
        return carry

    lax.fori_loop(0, n_chunks // NB, outer, 0)


def kernel(data, table):
    B0, S = data.shape
    V, D = table.shape
    B = B0 * S
    idx = data.reshape(B).astype(jnp.int32)

    b_per_w = B // NW
    n_chunks = b_per_w // CH
    assert b_per_w * NW == B and n_chunks * CH == b_per_w
    assert n_chunks % NB == 0

    mesh = plsc.VectorSubcoreMesh(core_axis_name="c", subcore_axis_name="s")
    run = functools.partial(
        pl.kernel,
        out_type=jax.ShapeDtypeStruct((B, D), jnp.float32),
        mesh=mesh,
        scratch_types=[
            pltpu.VMEM((b_per_w,), jnp.int32),
            pltpu.VMEM((NB, CH, D), jnp.float32),
            pltpu.SemaphoreType.DMA,
            pltpu.SemaphoreType.DMA,
        ],
        compiler_params=pltpu.CompilerParams(use_tc_tiling_on_sc=False),
    )(functools.partial(_gather_kernel, b_per_w, n_chunks, D))
    out = run(idx, table)
    return out.reshape(B0, S, D)
